# baseline (speedup 1.0000x reference)
import functools

import jax
import jax.numpy as jnp
from jax import lax
from jax.experimental import pallas as pl
from jax.experimental.pallas import tpu as pltpu
from jax.experimental.pallas import tpu_sc as plsc

N = 50000
D = 256
G = 64
L = 16
NC, NS = 2, 16
NW = NC * NS
CH = 1568
NV = CH // L
NPAD = NW * CH
PAD = NPAD - N
NEG = -1e30
B = 2000
NBLK = N // B

_mesh = plsc.VectorSubcoreMesh(core_axis_name="c", subcore_axis_name="s")


def _logits_body(x_ref, w_ref, l_ref):
    l_ref[...] = jnp.dot(x_ref[...], w_ref[...],
                         preferred_element_type=jnp.float32)


def _logits(x, W):
    return pl.pallas_call(
        _logits_body,
        grid=(NBLK,),
        in_specs=[
            pl.BlockSpec((B, D), lambda i: (i, 0)),
            pl.BlockSpec((D, 1), lambda i: (0, 0)),
        ],
        out_specs=pl.BlockSpec((B, 1), lambda i: (i, 0)),
        out_shape=jax.ShapeDtypeStruct((N, 1), jnp.float32),
    )(x, W)


def _tree_max(v, rots):
    for r in rots:
        v = jnp.maximum(v, jnp.take(v, r))
    return v


def _tree_sum(v, rots):
    for r in rots:
        v = v + jnp.take(v, r)
    return v


def _stats_body(l_hbm, seg_hbm, mp_hbm, dp_hbm, lb, segb, mrec, drec):
    w = lax.axis_index("s") * NC + lax.axis_index("c")
    base = w * CH
    pltpu.sync_copy(l_hbm.at[pl.ds(base, CH)], lb)
    pltpu.sync_copy(seg_hbm.at[pl.ds(base, CH)], segb)

    zeros = jnp.zeros((L,), jnp.float32)
    negv = jnp.full((L,), NEG, jnp.float32)
    for s in range(G):
        mrec[pl.ds(s * L, L)] = negv
        drec[pl.ds(s * L, L)] = zeros

    s_lo = segb[pl.ds(0, L)][0]
    s_hi = segb[pl.ds(CH - L, L)][L - 1]
    iota = lax.broadcasted_iota(jnp.int32, (L,), 0)
    rots = [(iota + r) % L for r in (8, 4, 2, 1)]

    def _seg(s, _):
        def _mstep(k, mv):
            sv = segb[pl.ds(k * L, L)]
            lv = lb[pl.ds(k * L, L)]
            return jnp.maximum(mv, jnp.where(sv == s, lv, NEG))

        mv = _tree_max(lax.fori_loop(0, NV, _mstep, negv), rots)

        def _dstep(k, dv):
            sv = segb[pl.ds(k * L, L)]
            lv = lb[pl.ds(k * L, L)]
            return dv + jnp.where(sv == s, jnp.exp(lv - mv), 0.0)

        dv = _tree_sum(lax.fori_loop(0, NV, _dstep, zeros), rots)

        mrec[pl.ds(s * L, L)] = mv
        drec[pl.ds(s * L, L)] = dv
        return 0

    lax.fori_loop(s_lo, s_hi + 1, _seg, 0)

    pltpu.sync_copy(mrec, mp_hbm.at[w])
    pltpu.sync_copy(drec, dp_hbm.at[w])


_stats = functools.partial(
    pl.kernel,
    out_type=(
        jax.ShapeDtypeStruct((NW, G * L), jnp.float32),
        jax.ShapeDtypeStruct((NW, G * L), jnp.float32),
    ),
    mesh=_mesh,
    scratch_types=[
        pltpu.VMEM((CH,), jnp.float32),
        pltpu.VMEM((CH,), jnp.int32),
        pltpu.VMEM((G * L,), jnp.float32),
        pltpu.VMEM((G * L,), jnp.float32),
    ],
)(_stats_body)


def _pool_body(x_ref, seg_ref, l_ref, mp_ref, dp_ref, out_ref,
               m_ref, di_ref, acc_ref):
    i = pl.program_id(0)

    @pl.when(i == 0)
    def _init():
        mp = mp_ref[...]
        dp = dp_ref[...]
        m = jnp.max(mp, axis=0, keepdims=True)
        d = jnp.sum(dp * jnp.exp(mp - m), axis=0, keepdims=True)
        m_ref[...] = m
        di_ref[...] = 1.0 / (d + 1e-16)
        acc_ref[...] = jnp.zeros(acc_ref.shape, jnp.float32)

    x_blk = x_ref[...]
    l = l_ref[...]
    seg = seg_ref[0, 0, :].reshape(B, 1)
    gids = lax.broadcasted_iota(jnp.int32, (B, G), 1)
    onehot_b = seg == gids
    m_row = jnp.max(jnp.where(onehot_b, m_ref[...], NEG), axis=1,
                    keepdims=True)
    p = jnp.exp(l - m_row)
    onehot = onehot_b.astype(jnp.float32)
    acc_ref[...] += lax.dot_general(onehot, p * x_blk, (((0,), (0,)), ((), ())),
                                    preferred_element_type=jnp.float32)

    @pl.when(i == pl.num_programs(0) - 1)
    def _finish():
        out_ref[...] = acc_ref[...] * di_ref[...].reshape(G, 1)


def _pool(x, seg3, l, mp, dp):
    return pl.pallas_call(
        _pool_body,
        grid=(NBLK,),
        in_specs=[
            pl.BlockSpec((B, D), lambda i: (i, 0)),
            pl.BlockSpec((1, 1, B), lambda i: (i, 0, 0)),
            pl.BlockSpec((B, 1), lambda i: (i, 0)),
            pl.BlockSpec((NW, G), lambda i: (0, 0)),
            pl.BlockSpec((NW, G), lambda i: (0, 0)),
        ],
        out_specs=pl.BlockSpec((G, D), lambda i: (0, 0)),
        out_shape=jax.ShapeDtypeStruct((G, D), jnp.float32),
        scratch_shapes=[
            pltpu.VMEM((1, G), jnp.float32),
            pltpu.VMEM((1, G), jnp.float32),
            pltpu.VMEM((G, D), jnp.float32),
        ],
    )(x, seg3, l, mp, dp)


def kernel(x, batch, W, b):
    del b
    seg = batch.astype(jnp.int32)
    l = _logits(x, W)
    lpad = jnp.concatenate(
        [l.reshape(N), jnp.full((PAD,), NEG, jnp.float32)])
    segpad = jnp.concatenate(
        [seg, jnp.full((PAD,), G - 1, jnp.int32)])
    mp_rec, dp_rec = _stats(lpad, segpad)
    mp = mp_rec.reshape(NW, G, L)[:, :, 0]
    dp = dp_rec.reshape(NW, G, L)[:, :, 0]
    seg3 = seg.reshape(NBLK, 1, B)
    return _pool(x, seg3, l, mp, dp)


# speedup vs baseline: 1.0624x; 1.0624x over previous
import functools

import jax
import jax.numpy as jnp
from jax import lax
from jax.experimental import pallas as pl
from jax.experimental.pallas import tpu as pltpu
from jax.experimental.pallas import tpu_sc as plsc

N = 50000
D = 256
G = 64
L = 16
NC, NS = 2, 16
NW = NC * NS
CH = 1568
NV = CH // L
K_OVL = (NW * CH - N) // L
NEG = -1e30
B = 2000
NBLK = N // B

_mesh = plsc.VectorSubcoreMesh(core_axis_name="c", subcore_axis_name="s")


def _logits_body(x_ref, w_ref, l_ref):
    l_ref[...] = jnp.dot(x_ref[...], w_ref[...],
                         preferred_element_type=jnp.float32)


def _logits(x, W):
    return pl.pallas_call(
        _logits_body,
        grid=(NBLK,),
        in_specs=[
            pl.BlockSpec((B, D), lambda i: (i, 0)),
            pl.BlockSpec((D, 1), lambda i: (0, 0)),
        ],
        out_specs=pl.BlockSpec((B, 1), lambda i: (i, 0)),
        out_shape=jax.ShapeDtypeStruct((N, 1), jnp.float32),
    )(x, W)


def _tree_max(v, rots):
    for r in rots:
        v = jnp.maximum(v, jnp.take(v, r))
    return v


def _tree_sum(v, rots):
    for r in rots:
        v = v + jnp.take(v, r)
    return v


def _stats_body(l_hbm, seg_hbm, mp_hbm, dp_hbm, lb, segb, mrec, drec):
    w = lax.axis_index("s") * NC + lax.axis_index("c")
    last = w == NW - 1
    base = jnp.where(last, N - CH, w * CH)
    k_lo = jnp.where(last, K_OVL, 0)
    pltpu.sync_copy(l_hbm.at[pl.ds(base, CH)], lb)
    pltpu.sync_copy(seg_hbm.at[pl.ds(base, CH)], segb)

    zeros = jnp.zeros((L,), jnp.float32)
    negv = jnp.full((L,), NEG, jnp.float32)
    for s in range(G):
        mrec[pl.ds(s * L, L)] = negv
        drec[pl.ds(s * L, L)] = zeros

    s_lo = segb[pl.ds(k_lo * L, L)][0]
    s_hi = segb[pl.ds(CH - L, L)][L - 1]
    iota = lax.broadcasted_iota(jnp.int32, (L,), 0)
    rots = [(iota + r) % L for r in (8, 4, 2, 1)]

    def _seg(s, _):
        def _mstep(k, mv):
            sv = segb[pl.ds(k * L, L)]
            lv = lb[pl.ds(k * L, L)]
            return jnp.maximum(mv, jnp.where(sv == s, lv, NEG))

        mv = _tree_max(lax.fori_loop(k_lo, NV, _mstep, negv), rots)

        def _dstep(k, dv):
            sv = segb[pl.ds(k * L, L)]
            lv = lb[pl.ds(k * L, L)]
            return dv + jnp.where(sv == s, jnp.exp(lv - mv), 0.0)

        dv = _tree_sum(lax.fori_loop(k_lo, NV, _dstep, zeros), rots)

        mrec[pl.ds(s * L, L)] = mv
        drec[pl.ds(s * L, L)] = dv
        return 0

    lax.fori_loop(s_lo, s_hi + 1, _seg, 0)

    pltpu.sync_copy(mrec, mp_hbm.at[w])
    pltpu.sync_copy(drec, dp_hbm.at[w])


_stats = functools.partial(
    pl.kernel,
    out_type=(
        jax.ShapeDtypeStruct((NW, G * L), jnp.float32),
        jax.ShapeDtypeStruct((NW, G * L), jnp.float32),
    ),
    mesh=_mesh,
    scratch_types=[
        pltpu.VMEM((CH,), jnp.float32),
        pltpu.VMEM((CH,), jnp.int32),
        pltpu.VMEM((G * L,), jnp.float32),
        pltpu.VMEM((G * L,), jnp.float32),
    ],
)(_stats_body)


def _pool_body(x_ref, seg_ref, l_ref, mp_ref, dp_ref, out_ref,
               m_ref, di_ref, acc_ref):
    i = pl.program_id(0)

    @pl.when(i == 0)
    def _init():
        sel = (lax.broadcasted_iota(jnp.int32, (G * L, G), 0) ==
               lax.broadcasted_iota(jnp.int32, (G * L, G), 1) * L
               ).astype(jnp.float32)
        mp = jnp.dot(mp_ref[...], sel,
                     preferred_element_type=jnp.float32)
        dp = jnp.dot(dp_ref[...], sel,
                     preferred_element_type=jnp.float32)
        m = jnp.max(mp, axis=0, keepdims=True)
        d = jnp.sum(dp * jnp.exp(mp - m), axis=0, keepdims=True)
        m_ref[...] = m
        di_ref[...] = 1.0 / (d + 1e-16)
        acc_ref[...] = jnp.zeros(acc_ref.shape, jnp.float32)

    x_blk = x_ref[...]
    l = l_ref[...]
    seg = seg_ref[0, 0, :].reshape(B, 1)
    gids = lax.broadcasted_iota(jnp.int32, (B, G), 1)
    onehot = (seg == gids).astype(jnp.float32)
    m_row = jnp.dot(onehot, m_ref[...].reshape(G, 1),
                    preferred_element_type=jnp.float32)
    p = jnp.exp(l - m_row)
    acc_ref[...] += lax.dot_general(onehot, p * x_blk, (((0,), (0,)), ((), ())),
                                    preferred_element_type=jnp.float32)

    @pl.when(i == pl.num_programs(0) - 1)
    def _finish():
        out_ref[...] = acc_ref[...] * di_ref[...].reshape(G, 1)


def _pool(x, seg3, l, mp, dp):
    return pl.pallas_call(
        _pool_body,
        grid=(NBLK,),
        in_specs=[
            pl.BlockSpec((B, D), lambda i: (i, 0)),
            pl.BlockSpec((1, 1, B), lambda i: (i, 0, 0)),
            pl.BlockSpec((B, 1), lambda i: (i, 0)),
            pl.BlockSpec((NW, G * L), lambda i: (0, 0)),
            pl.BlockSpec((NW, G * L), lambda i: (0, 0)),
        ],
        out_specs=pl.BlockSpec((G, D), lambda i: (0, 0)),
        out_shape=jax.ShapeDtypeStruct((G, D), jnp.float32),
        scratch_shapes=[
            pltpu.VMEM((1, G), jnp.float32),
            pltpu.VMEM((1, G), jnp.float32),
            pltpu.VMEM((G, D), jnp.float32),
        ],
    )(x, seg3, l, mp, dp)


def kernel(x, batch, W, b):
    del b
    seg = batch.astype(jnp.int32)
    l = _logits(x, W)
    mp_rec, dp_rec = _stats(l.reshape(N), seg)
    seg3 = seg.reshape(NBLK, 1, B)
    return _pool(x, seg3, l, mp_rec, dp_rec)


# speedup vs baseline: 1.0643x; 1.0018x over previous
import functools

import jax
import jax.numpy as jnp
from jax import lax
from jax.experimental import pallas as pl
from jax.experimental.pallas import tpu as pltpu
from jax.experimental.pallas import tpu_sc as plsc

N = 50000
D = 256
G = 64
L = 16
NC, NS = 2, 16
NW = NC * NS
CH = 1568
NV = CH // L
K_OVL = (NW * CH - N) // L
NEG = -1e30
B = 2000
NBLK = N // B

_mesh = plsc.VectorSubcoreMesh(core_axis_name="c", subcore_axis_name="s")


def _logits_body(x_ref, w_ref, l_ref):
    l_ref[...] = jnp.dot(x_ref[...], w_ref[...],
                         preferred_element_type=jnp.float32)


def _logits(x, W):
    return pl.pallas_call(
        _logits_body,
        grid=(NBLK,),
        in_specs=[
            pl.BlockSpec((B, D), lambda i: (i, 0)),
            pl.BlockSpec((D, 1), lambda i: (0, 0)),
        ],
        out_specs=pl.BlockSpec((B, 1), lambda i: (i, 0)),
        out_shape=jax.ShapeDtypeStruct((N, 1), jnp.float32),
    )(x, W)


def _tree_max(v, rots):
    for r in rots:
        v = jnp.maximum(v, jnp.take(v, r))
    return v


def _tree_sum(v, rots):
    for r in rots:
        v = v + jnp.take(v, r)
    return v


def _stats_body(l_hbm, seg_hbm, mp_hbm, dp_hbm, lb, segb, mrec, drec):
    w = lax.axis_index("s") * NC + lax.axis_index("c")
    last = w == NW - 1
    base = jnp.where(last, N - CH, w * CH)
    k_lo = jnp.where(last, K_OVL, 0)
    pltpu.sync_copy(l_hbm.at[pl.ds(base, CH)], lb)
    pltpu.sync_copy(seg_hbm.at[pl.ds(base, CH)], segb)

    zeros = jnp.zeros((L,), jnp.float32)
    negv = jnp.full((L,), NEG, jnp.float32)
    for s in range(G):
        mrec[pl.ds(s * L, L)] = negv
        drec[pl.ds(s * L, L)] = zeros

    s_lo = segb[pl.ds(k_lo * L, L)][0]
    s_hi = segb[pl.ds(CH - L, L)][L - 1]
    iota = lax.broadcasted_iota(jnp.int32, (L,), 0)
    rots = [(iota + r) % L for r in (8, 4, 2, 1)]

    def _seg(s, _):
        def _mstep(k, mv):
            sv = segb[pl.ds(k * L, L)]
            lv = lb[pl.ds(k * L, L)]
            return jnp.maximum(mv, jnp.where(sv == s, lv, NEG))

        mv = _tree_max(lax.fori_loop(k_lo, NV, _mstep, negv), rots)

        def _dstep(k, dv):
            sv = segb[pl.ds(k * L, L)]
            lv = lb[pl.ds(k * L, L)]
            return dv + jnp.where(sv == s, jnp.exp(lv - mv), 0.0)

        dv = _tree_sum(lax.fori_loop(k_lo, NV, _dstep, zeros), rots)

        mrec[pl.ds(s * L, L)] = mv
        drec[pl.ds(s * L, L)] = dv
        return 0

    lax.fori_loop(s_lo, s_hi + 1, _seg, 0)

    pltpu.sync_copy(mrec, mp_hbm.at[w])
    pltpu.sync_copy(drec, dp_hbm.at[w])


_stats = functools.partial(
    pl.kernel,
    out_type=(
        jax.ShapeDtypeStruct((NW, G * L), jnp.float32),
        jax.ShapeDtypeStruct((NW, G * L), jnp.float32),
    ),
    mesh=_mesh,
    scratch_types=[
        pltpu.VMEM((CH,), jnp.float32),
        pltpu.VMEM((CH,), jnp.int32),
        pltpu.VMEM((G * L,), jnp.float32),
        pltpu.VMEM((G * L,), jnp.float32),
    ],
)(_stats_body)


def _pool_body(x_ref, seg_ref, l_ref, mp_ref, dp_ref, out_ref,
               m_ref, di_ref, acc_ref):
    i = pl.program_id(0)

    @pl.when(i == 0)
    def _init():
        sel = (lax.broadcasted_iota(jnp.int32, (G * L, G), 0) ==
               lax.broadcasted_iota(jnp.int32, (G * L, G), 1) * L
               ).astype(jnp.float32)
        mp = jnp.dot(mp_ref[...], sel,
                     preferred_element_type=jnp.float32)
        dp = jnp.dot(dp_ref[...], sel,
                     preferred_element_type=jnp.float32)
        m = jnp.max(mp, axis=0, keepdims=True)
        d = jnp.sum(dp * jnp.exp(jnp.minimum(mp - m, 0.0)), axis=0,
                    keepdims=True)
        m_ref[...] = m
        di_ref[...] = 1.0 / (d + 1e-16)
        acc_ref[...] = jnp.zeros(acc_ref.shape, jnp.float32)

    x_blk = x_ref[...]
    l = l_ref[...]
    seg = seg_ref[0, 0, :].reshape(B, 1)
    gids = lax.broadcasted_iota(jnp.int32, (B, G), 1)
    ez = jnp.exp(l - m_ref[...])
    onehot_p = jnp.where(seg == gids, ez, 0.0)
    acc_ref[...] += lax.dot_general(onehot_p, x_blk, (((0,), (0,)), ((), ())),
                                    preferred_element_type=jnp.float32)

    @pl.when(i == pl.num_programs(0) - 1)
    def _finish():
        out_ref[...] = acc_ref[...] * di_ref[...].reshape(G, 1)


def _pool(x, seg3, l, mp, dp):
    return pl.pallas_call(
        _pool_body,
        grid=(NBLK,),
        in_specs=[
            pl.BlockSpec((B, D), lambda i: (i, 0)),
            pl.BlockSpec((1, 1, B), lambda i: (i, 0, 0)),
            pl.BlockSpec((B, 1), lambda i: (i, 0)),
            pl.BlockSpec((NW, G * L), lambda i: (0, 0)),
            pl.BlockSpec((NW, G * L), lambda i: (0, 0)),
        ],
        out_specs=pl.BlockSpec((G, D), lambda i: (0, 0)),
        out_shape=jax.ShapeDtypeStruct((G, D), jnp.float32),
        scratch_shapes=[
            pltpu.VMEM((1, G), jnp.float32),
            pltpu.VMEM((1, G), jnp.float32),
            pltpu.VMEM((G, D), jnp.float32),
        ],
    )(x, seg3, l, mp, dp)


def kernel(x, batch, W, b):
    del b
    seg = batch.astype(jnp.int32)
    l = _logits(x, W)
    mp_rec, dp_rec = _stats(l.reshape(N), seg)
    seg3 = seg.reshape(NBLK, 1, B)
    return _pool(x, seg3, l, mp_rec, dp_rec)
